# baseline (speedup 1.0000x reference)
import functools

import jax
import jax.numpy as jnp
from jax import lax
from jax.experimental import pallas as pl
from jax.experimental.pallas import tpu as pltpu
from jax.experimental.pallas import tpu_sc as plsc

B, D, H, E = 2048, 1024, 2048, 8
R = 256
NMB = (B * 2) // R + E
NPAD = NMB * R
HBLK = 1024
NHB = H // HBLK

NC, NS = 2, 16
NW = NC * NS
TPW = B // NW
CH = 16
LN = 16


def _cumsum_lanes(a):
    n = 1
    w = a.shape[1]
    while n < w:
        pad = jnp.zeros((a.shape[0], n), a.dtype)
        a = a + jnp.concatenate([pad, a[:, : w - n]], axis=1)
        n *= 2
    return a


def _cumsum_sublanes(a):
    n = 1
    h = a.shape[0]
    while n < h:
        pad = jnp.zeros((n, a.shape[1]), a.dtype)
        a = a + jnp.concatenate([pad, a[: h - n, :]], axis=0)
        n *= 2
    return a


def _router_body(xt_ref, gw_ref, gb_ref,
                 pos0_ref, pos1_ref, s0_ref, s1_ref, bexp_ref, used_ref):
    gw = gw_ref[...].astype(jnp.bfloat16)
    xt = xt_ref[...].astype(jnp.bfloat16)
    logits = lax.dot_general(
        gw, xt, (((1,), (0,)), ((), ())),
        preferred_element_type=jnp.float32)
    logits = logits + gb_ref[...]
    m = jnp.max(logits, axis=0, keepdims=True)
    p = jnp.exp(logits - m)
    p = p / jnp.sum(p, axis=0, keepdims=True)

    eidx = lax.broadcasted_iota(jnp.int32, (E, B), 0).astype(jnp.float32)
    s1 = jnp.max(p, axis=0, keepdims=True)
    i1 = jnp.min(jnp.where(p == s1, eidx, jnp.float32(E)), axis=0, keepdims=True)
    oh0 = eidx == i1
    p2 = jnp.where(oh0, jnp.float32(-1.0), p)
    s2 = jnp.max(p2, axis=0, keepdims=True)
    i2 = jnp.min(jnp.where(p2 == s2, eidx, jnp.float32(E)), axis=0, keepdims=True)
    oh1 = eidx == i2

    c0 = _cumsum_lanes(oh0.astype(jnp.float32))
    c1 = _cumsum_lanes(oh1.astype(jnp.float32))
    t0 = c0[:, B - 1:B]
    t1 = c1[:, B - 1:B]
    g = t0 + t1
    gp = jnp.floor((g + (R - 1)) / R) * R
    base_incl = _cumsum_sublanes(gp)
    base = base_incl - gp

    pos0 = jnp.sum(jnp.where(oh0, base + c0 - 1.0, 0.0), axis=0, keepdims=True)
    pos1 = jnp.sum(jnp.where(oh1, base + t0 + c1 - 1.0, 0.0), axis=0, keepdims=True)
    pos0_ref[...] = pos0.astype(jnp.int32)
    pos1_ref[...] = pos1.astype(jnp.int32)
    s0_ref[...] = s1
    s1_ref[...] = s2

    total = base_incl[E - 1:E, :]
    usedb = total / R
    used_ref[...] = usedb.astype(jnp.int32)
    bi = lax.broadcasted_iota(jnp.int32, (1, NMB), 1).astype(jnp.float32)
    bi = jnp.minimum(bi, usedb - 1.0)
    bexp = jnp.sum((base <= bi * R).astype(jnp.float32), axis=0, keepdims=True) - 1.0
    bexp_ref[...] = bexp.astype(jnp.int32)


def _router(xt, gw, gb):
    return pl.pallas_call(
        _router_body,
        out_shape=(
            jax.ShapeDtypeStruct((1, B), jnp.int32),
            jax.ShapeDtypeStruct((1, B), jnp.int32),
            jax.ShapeDtypeStruct((1, B), jnp.float32),
            jax.ShapeDtypeStruct((1, B), jnp.float32),
            jax.ShapeDtypeStruct((1, NMB), jnp.int32),
            jax.ShapeDtypeStruct((1, 1), jnp.int32),
        ),
    )(xt, gw, gb)


def _gemm_body(bexp_ref, used_ref, xs_ref, wv_ref, wg_ref, ys_ref):
    m = pl.program_id(1)

    @pl.when(m < used_ref[0])
    def _():
        xb = xs_ref[...].astype(jnp.bfloat16)
        wv = wv_ref[0].astype(jnp.bfloat16)
        wg = wg_ref[0].astype(jnp.bfloat16)
        v = jnp.dot(xb, wv, preferred_element_type=jnp.float32)
        g = jnp.dot(xb, wg, preferred_element_type=jnp.float32)
        ys_ref[...] = v * (1.0 / (1.0 + jnp.exp(-g)))


def _grouped_gemm(bexp, used, xs, wv, wg):
    grid_spec = pltpu.PrefetchScalarGridSpec(
        num_scalar_prefetch=2,
        grid=(NHB, NMB),
        in_specs=[
            pl.BlockSpec((R, D), lambda h, m, be, us: (m, 0)),
            pl.BlockSpec((1, D, HBLK), lambda h, m, be, us: (be[m], 0, h)),
            pl.BlockSpec((1, D, HBLK), lambda h, m, be, us: (be[m], 0, h)),
        ],
        out_specs=pl.BlockSpec((R, HBLK), lambda h, m, be, us: (m, h)),
    )
    return pl.pallas_call(
        _gemm_body,
        grid_spec=grid_spec,
        out_shape=jax.ShapeDtypeStruct((NPAD, H), jnp.float32),
        compiler_params=pltpu.CompilerParams(
            dimension_semantics=("arbitrary", "arbitrary")),
    )(bexp, used, xs, wv, wg)


@functools.lru_cache(maxsize=1)
def _sc_kernels():
    mesh = plsc.VectorSubcoreMesh(core_axis_name="c", subcore_axis_name="s")

    @functools.partial(
        pl.kernel,
        mesh=mesh,
        out_type=jax.ShapeDtypeStruct((NPAD, D), jnp.float32),
        scratch_types=[
            pltpu.VMEM((TPW, D), jnp.float32),
            pltpu.VMEM((TPW,), jnp.int32),
            pltpu.VMEM((TPW,), jnp.int32),
            pltpu.SemaphoreType.DMA,
        ],
    )
    def scatter(x_hbm, pos0_hbm, pos1_hbm, xs_hbm, rows_v, i0_v, i1_v, sem):
        wid = lax.axis_index("s") * NC + lax.axis_index("c")
        base = wid * TPW
        pltpu.sync_copy(x_hbm.at[pl.ds(base, TPW)], rows_v)
        pltpu.sync_copy(pos0_hbm.at[pl.ds(base, TPW)], i0_v)
        pltpu.sync_copy(pos1_hbm.at[pl.ds(base, TPW)], i1_v)
        c0 = pltpu.async_copy(rows_v, xs_hbm.at[i0_v], sem)
        c1 = pltpu.async_copy(rows_v, xs_hbm.at[i1_v], sem)
        c0.wait()
        c1.wait()

    @functools.partial(
        pl.kernel,
        mesh=mesh,
        out_type=jax.ShapeDtypeStruct((B, H), jnp.float32),
        scratch_types=[
            pltpu.VMEM((CH, H), jnp.float32),
            pltpu.VMEM((CH, H), jnp.float32),
            pltpu.VMEM((CH, H), jnp.float32),
            pltpu.VMEM((CH,), jnp.int32),
            pltpu.VMEM((CH,), jnp.int32),
            pltpu.VMEM((CH,), jnp.float32),
            pltpu.VMEM((CH,), jnp.float32),
            pltpu.SemaphoreType.DMA,
        ],
    )
    def combine(ys_hbm, pos0_hbm, pos1_hbm, s0_hbm, s1_hbm, out_hbm,
                ya_v, yb_v, ob_v, ia_v, ib_v, sa_v, sb_v, sem):
        wid = lax.axis_index("s") * NC + lax.axis_index("c")
        base = wid * TPW
        for cc in range(TPW // CH):
            tb = base + cc * CH
            pltpu.sync_copy(pos0_hbm.at[pl.ds(tb, CH)], ia_v)
            pltpu.sync_copy(pos1_hbm.at[pl.ds(tb, CH)], ib_v)
            pltpu.sync_copy(s0_hbm.at[pl.ds(tb, CH)], sa_v)
            pltpu.sync_copy(s1_hbm.at[pl.ds(tb, CH)], sb_v)
            ca = pltpu.async_copy(ys_hbm.at[ia_v], ya_v, sem)
            cb = pltpu.async_copy(ys_hbm.at[ib_v], yb_v, sem)
            ca.wait()
            cb.wait()
            sa = sa_v[...]
            sb = sb_v[...]
            sas = [sa[t] for t in range(CH)]
            sbs = [sb[t] for t in range(CH)]

            def body(hc, _):
                sl = pl.ds(hc * LN, LN)
                for t in range(CH):
                    ob_v[t, sl] = sas[t] * ya_v[t, sl] + sbs[t] * yb_v[t, sl]
                return 0

            lax.fori_loop(0, H // LN, body, 0)
            pltpu.sync_copy(ob_v, out_hbm.at[pl.ds(tb, CH)])

    return scatter, combine


def kernel(x, expert_weights_v, expert_weights_g, gate_w, gate_b):
    xt = x.T
    gb = gate_b.reshape(E, 1)
    pos0, pos1, s0, s1, bexp, used = _router(xt, gate_w, gb)
    pos0 = pos0.reshape(B)
    pos1 = pos1.reshape(B)
    scatter, combine = _sc_kernels()
    xs = scatter(x, pos0, pos1)
    ys = _grouped_gemm(bexp.reshape(NMB), used.reshape(1),
                       xs, expert_weights_v, expert_weights_g)
    return combine(ys, pos0, pos1, s0.reshape(B), s1.reshape(B))


# speedup vs baseline: 1.2316x; 1.2316x over previous
import functools

import jax
import jax.numpy as jnp
from jax import lax
from jax.experimental import pallas as pl
from jax.experimental.pallas import tpu as pltpu
from jax.experimental.pallas import tpu_sc as plsc

B, D, H, E = 2048, 1024, 2048, 8
R = 256
NMB = (B * 2) // R + E
NPAD = NMB * R
HBLK = 1024
NHB = H // HBLK

NC, NS = 2, 16
NW = NC * NS
TPW = B // NW
CH = 16
LN = 16


def _cumsum_lanes(a):
    n = 1
    w = a.shape[1]
    while n < w:
        pad = jnp.zeros((a.shape[0], n), a.dtype)
        a = a + jnp.concatenate([pad, a[:, : w - n]], axis=1)
        n *= 2
    return a


def _cumsum_sublanes(a):
    n = 1
    h = a.shape[0]
    while n < h:
        pad = jnp.zeros((n, a.shape[1]), a.dtype)
        a = a + jnp.concatenate([pad, a[: h - n, :]], axis=0)
        n *= 2
    return a


def _router_body(xt_ref, gw_ref, gb_ref,
                 pos0_ref, pos1_ref, s0_ref, s1_ref, bexp_ref, used_ref):
    gw = gw_ref[...].astype(jnp.bfloat16)
    xt = xt_ref[...].astype(jnp.bfloat16)
    logits = lax.dot_general(
        gw, xt, (((1,), (0,)), ((), ())),
        preferred_element_type=jnp.float32)
    logits = logits + gb_ref[...]
    m = jnp.max(logits, axis=0, keepdims=True)
    p = jnp.exp(logits - m)
    p = p / jnp.sum(p, axis=0, keepdims=True)

    eidx = lax.broadcasted_iota(jnp.int32, (E, B), 0).astype(jnp.float32)
    s1 = jnp.max(p, axis=0, keepdims=True)
    i1 = jnp.min(jnp.where(p == s1, eidx, jnp.float32(E)), axis=0, keepdims=True)
    oh0 = eidx == i1
    p2 = jnp.where(oh0, jnp.float32(-1.0), p)
    s2 = jnp.max(p2, axis=0, keepdims=True)
    i2 = jnp.min(jnp.where(p2 == s2, eidx, jnp.float32(E)), axis=0, keepdims=True)
    oh1 = eidx == i2

    c0 = _cumsum_lanes(oh0.astype(jnp.float32))
    c1 = _cumsum_lanes(oh1.astype(jnp.float32))
    t0 = c0[:, B - 1:B]
    t1 = c1[:, B - 1:B]
    g = t0 + t1
    gp = jnp.floor((g + (R - 1)) / R) * R
    base_incl = _cumsum_sublanes(gp)
    base = base_incl - gp

    pos0 = jnp.sum(jnp.where(oh0, base + c0 - 1.0, 0.0), axis=0, keepdims=True)
    pos1 = jnp.sum(jnp.where(oh1, base + t0 + c1 - 1.0, 0.0), axis=0, keepdims=True)
    pos0_ref[...] = pos0.astype(jnp.int32)
    pos1_ref[...] = pos1.astype(jnp.int32)
    s0_ref[...] = s1
    s1_ref[...] = s2

    total = base_incl[E - 1:E, :]
    usedb = total / R
    used_ref[...] = usedb.astype(jnp.int32)
    bi = lax.broadcasted_iota(jnp.int32, (1, NMB), 1).astype(jnp.float32)
    bi = jnp.minimum(bi, usedb - 1.0)
    bexp = jnp.sum((base <= bi * R).astype(jnp.float32), axis=0, keepdims=True) - 1.0
    bexp_ref[...] = bexp.astype(jnp.int32)


def _router(xt, gw, gb):
    return pl.pallas_call(
        _router_body,
        out_shape=(
            jax.ShapeDtypeStruct((1, B), jnp.int32),
            jax.ShapeDtypeStruct((1, B), jnp.int32),
            jax.ShapeDtypeStruct((1, B), jnp.float32),
            jax.ShapeDtypeStruct((1, B), jnp.float32),
            jax.ShapeDtypeStruct((1, NMB), jnp.int32),
            jax.ShapeDtypeStruct((1, 1), jnp.int32),
        ),
    )(xt, gw, gb)


def _gemm_body(bexp_ref, used_ref, xs_ref, wv_ref, wg_ref, ylo_ref, yhi_ref):
    m = pl.program_id(0)

    @pl.when(m < used_ref[0])
    def _():
        xb = xs_ref[...]
        wv = wv_ref[0]
        wg = wg_ref[0]
        v = jnp.dot(xb, wv, preferred_element_type=jnp.float32)
        g = jnp.dot(xb, wg, preferred_element_type=jnp.float32)
        act = v * (1.0 / (1.0 + jnp.exp(-g)))
        ylo_ref[...] = act[:, :HBLK]
        yhi_ref[...] = act[:, HBLK:]


def _grouped_gemm(bexp, used, xs, wv, wg):
    grid_spec = pltpu.PrefetchScalarGridSpec(
        num_scalar_prefetch=2,
        grid=(NMB,),
        in_specs=[
            pl.BlockSpec((R, D), lambda m, be, us: (m, 0)),
            pl.BlockSpec((1, D, H), lambda m, be, us: (be[m], 0, 0)),
            pl.BlockSpec((1, D, H), lambda m, be, us: (be[m], 0, 0)),
        ],
        out_specs=[
            pl.BlockSpec((R, HBLK), lambda m, be, us: (m, 0)),
            pl.BlockSpec((R, HBLK), lambda m, be, us: (m, 0)),
        ],
    )
    return pl.pallas_call(
        _gemm_body,
        grid_spec=grid_spec,
        out_shape=(jax.ShapeDtypeStruct((NPAD, HBLK), jnp.float32),
                   jax.ShapeDtypeStruct((NPAD, HBLK), jnp.float32)),
        compiler_params=pltpu.CompilerParams(
            dimension_semantics=("arbitrary",)),
    )(bexp, used, xs, wv, wg)


@functools.lru_cache(maxsize=1)
def _sc_kernels():
    mesh = plsc.VectorSubcoreMesh(core_axis_name="c", subcore_axis_name="s")

    @functools.partial(
        pl.kernel,
        mesh=mesh,
        out_type=jax.ShapeDtypeStruct((NPAD, D), jnp.float32),
        scratch_types=[
            pltpu.VMEM((TPW, D), jnp.float32),
            pltpu.VMEM((TPW,), jnp.int32),
            pltpu.VMEM((TPW,), jnp.int32),
            pltpu.SemaphoreType.DMA,
        ],
    )
    def scatter(x_hbm, pos0_hbm, pos1_hbm, xs_hbm, rows_v, i0_v, i1_v, sem):
        wid = lax.axis_index("s") * NC + lax.axis_index("c")
        base = wid * TPW
        pltpu.sync_copy(x_hbm.at[pl.ds(base, TPW)], rows_v)
        pltpu.sync_copy(pos0_hbm.at[pl.ds(base, TPW)], i0_v)
        pltpu.sync_copy(pos1_hbm.at[pl.ds(base, TPW)], i1_v)
        c0 = pltpu.async_copy(rows_v, xs_hbm.at[i0_v], sem)
        c1 = pltpu.async_copy(rows_v, xs_hbm.at[i1_v], sem)
        c0.wait()
        c1.wait()

    @functools.partial(
        pl.kernel,
        mesh=mesh,
        out_type=jax.ShapeDtypeStruct((B, H), jnp.float32),
        scratch_types=[
            pltpu.VMEM((2, CH, HBLK), jnp.float32),
            pltpu.VMEM((2, CH, HBLK), jnp.float32),
            pltpu.VMEM((CH, H), jnp.float32),
            pltpu.VMEM((2, CH), jnp.int32),
            pltpu.VMEM((2, CH), jnp.int32),
            pltpu.VMEM((CH,), jnp.float32),
            pltpu.VMEM((CH,), jnp.float32),
            pltpu.SemaphoreType.DMA,
            pltpu.SemaphoreType.DMA,
        ],
    )
    def combine(ylo_hbm, yhi_hbm, pos0_hbm, pos1_hbm, s0_hbm, s1_hbm, out_hbm,
                ya_v, yb_v, ob_v, ia_v, ib_v, sa_v, sb_v, sem0, sem1):
        wid = lax.axis_index("s") * NC + lax.axis_index("c")
        base = wid * TPW
        NCH = TPW // CH
        halves = (ylo_hbm, yhi_hbm)
        sems = (sem0, sem1)
        steps = [(cc, h) for cc in range(NCH) for h in range(2)]

        def load_idx(cc):
            tb = base + cc * CH
            pltpu.sync_copy(pos0_hbm.at[pl.ds(tb, CH)], ia_v.at[cc % 2])
            pltpu.sync_copy(pos1_hbm.at[pl.ds(tb, CH)], ib_v.at[cc % 2])

        def fire(cc, h, buf):
            sem = sems[buf]
            ca = pltpu.async_copy(halves[h].at[ia_v.at[cc % 2]],
                                  ya_v.at[buf], sem)
            cb = pltpu.async_copy(halves[h].at[ib_v.at[cc % 2]],
                                  yb_v.at[buf], sem)
            return ca, cb

        load_idx(0)
        pend = fire(0, 0, 0)
        for s, (cc, h) in enumerate(steps):
            buf = s % 2
            tb = base + cc * CH
            if h == 0:
                pltpu.sync_copy(s0_hbm.at[pl.ds(tb, CH)], sa_v)
                pltpu.sync_copy(s1_hbm.at[pl.ds(tb, CH)], sb_v)
                sa = sa_v[...]
                sb = sb_v[...]
                sas = [sa[t] for t in range(CH)]
                sbs = [sb[t] for t in range(CH)]
            if s + 1 < len(steps):
                cc2, h2 = steps[s + 1]
                if h2 == 0:
                    load_idx(cc2)
                nxt = fire(cc2, h2, (s + 1) % 2)
            pend[0].wait()
            pend[1].wait()

            def body(hc, _):
                sl = pl.ds(hc * LN, LN)
                ol = pl.ds(h * HBLK + hc * LN, LN)
                for t in range(CH):
                    ob_v[t, ol] = (sas[t] * ya_v[buf, t, sl]
                                   + sbs[t] * yb_v[buf, t, sl])
                return 0

            lax.fori_loop(0, HBLK // LN, body, 0)
            if h == 1:
                pltpu.sync_copy(ob_v, out_hbm.at[pl.ds(tb, CH)])
            if s + 1 < len(steps):
                pend = nxt

    return scatter, combine


def kernel(x, expert_weights_v, expert_weights_g, gate_w, gate_b):
    xt = x.T
    gb = gate_b.reshape(E, 1)
    pos0, pos1, s0, s1, bexp, used = _router(xt, gate_w, gb)
    pos0 = pos0.reshape(B)
    pos1 = pos1.reshape(B)
    scatter, combine = _sc_kernels()
    xs = scatter(x, pos0, pos1)
    ys_lo, ys_hi = _grouped_gemm(bexp.reshape(NMB), used.reshape(1),
                                 xs, expert_weights_v, expert_weights_g)
    return combine(ys_lo, ys_hi, pos0, pos1, s0.reshape(B), s1.reshape(B))


# speedup vs baseline: 1.2364x; 1.0039x over previous
import functools

import jax
import jax.numpy as jnp
from jax import lax
from jax.experimental import pallas as pl
from jax.experimental.pallas import tpu as pltpu
from jax.experimental.pallas import tpu_sc as plsc

B, D, H, E = 2048, 1024, 2048, 8
R = 256
NMB = (B * 2) // R + E
NPAD = NMB * R
HBLK = 1024
NHB = H // HBLK
HQ = HBLK // 2

NC, NS = 2, 16
NW = NC * NS
TPW = B // NW
CH = 16
LN = 16


def _cumsum_lanes(a):
    n = 1
    w = a.shape[1]
    while n < w:
        pad = jnp.zeros((a.shape[0], n), a.dtype)
        a = a + jnp.concatenate([pad, a[:, : w - n]], axis=1)
        n *= 2
    return a


def _cumsum_sublanes(a):
    n = 1
    h = a.shape[0]
    while n < h:
        pad = jnp.zeros((n, a.shape[1]), a.dtype)
        a = a + jnp.concatenate([pad, a[: h - n, :]], axis=0)
        n *= 2
    return a


def _router_body(xt_ref, gw_ref, gb_ref,
                 pos0_ref, pos1_ref, s0_ref, s1_ref, bexp_ref, used_ref):
    gw = gw_ref[...].astype(jnp.bfloat16)
    xt = xt_ref[...].astype(jnp.bfloat16)
    logits = lax.dot_general(
        gw, xt, (((1,), (0,)), ((), ())),
        preferred_element_type=jnp.float32)
    logits = logits + gb_ref[...]
    m = jnp.max(logits, axis=0, keepdims=True)
    p = jnp.exp(logits - m)
    p = p / jnp.sum(p, axis=0, keepdims=True)

    eidx = lax.broadcasted_iota(jnp.int32, (E, B), 0).astype(jnp.float32)
    s1 = jnp.max(p, axis=0, keepdims=True)
    i1 = jnp.min(jnp.where(p == s1, eidx, jnp.float32(E)), axis=0, keepdims=True)
    oh0 = eidx == i1
    p2 = jnp.where(oh0, jnp.float32(-1.0), p)
    s2 = jnp.max(p2, axis=0, keepdims=True)
    i2 = jnp.min(jnp.where(p2 == s2, eidx, jnp.float32(E)), axis=0, keepdims=True)
    oh1 = eidx == i2

    c0 = _cumsum_lanes(oh0.astype(jnp.float32))
    c1 = _cumsum_lanes(oh1.astype(jnp.float32))
    t0 = c0[:, B - 1:B]
    t1 = c1[:, B - 1:B]
    g = t0 + t1
    gp = jnp.floor((g + (R - 1)) / R) * R
    base_incl = _cumsum_sublanes(gp)
    base = base_incl - gp

    pos0 = jnp.sum(jnp.where(oh0, base + c0 - 1.0, 0.0), axis=0, keepdims=True)
    pos1 = jnp.sum(jnp.where(oh1, base + t0 + c1 - 1.0, 0.0), axis=0, keepdims=True)
    pos0_ref[...] = pos0.astype(jnp.int32)
    pos1_ref[...] = pos1.astype(jnp.int32)
    s0_ref[...] = s1
    s1_ref[...] = s2

    total = base_incl[E - 1:E, :]
    usedb = total / R
    used_ref[...] = usedb.astype(jnp.int32)
    bi = lax.broadcasted_iota(jnp.int32, (1, NMB), 1).astype(jnp.float32)
    bi = jnp.minimum(bi, usedb - 1.0)
    bexp = jnp.sum((base <= bi * R).astype(jnp.float32), axis=0, keepdims=True) - 1.0
    bexp_ref[...] = bexp.astype(jnp.int32)


def _router(xt, gw, gb):
    return pl.pallas_call(
        _router_body,
        out_shape=(
            jax.ShapeDtypeStruct((1, B), jnp.int32),
            jax.ShapeDtypeStruct((1, B), jnp.int32),
            jax.ShapeDtypeStruct((1, B), jnp.float32),
            jax.ShapeDtypeStruct((1, B), jnp.float32),
            jax.ShapeDtypeStruct((1, NMB), jnp.int32),
            jax.ShapeDtypeStruct((1, 1), jnp.int32),
        ),
    )(xt, gw, gb)


def _rne_hi16(a):
    u = lax.bitcast_convert_type(a, jnp.int32)
    r = u + jnp.int32(0x7FFF) + ((u >> 16) & jnp.int32(1))
    return r & jnp.int32(-65536)


def _gemm_body(bexp_ref, used_ref, xs_ref, wv_ref, wg_ref, ylo_ref, yhi_ref):
    m = pl.program_id(0)

    @pl.when(m < used_ref[0])
    def _():
        xb = xs_ref[...]
        wv = wv_ref[0]
        wg = wg_ref[0]
        v = jnp.dot(xb, wv, preferred_element_type=jnp.float32)
        g = jnp.dot(xb, wg, preferred_element_type=jnp.float32)
        act = v * (1.0 / (1.0 + jnp.exp(-g)))
        for h, ref in ((0, ylo_ref), (1, yhi_ref)):
            blk = act[:, h * HBLK:(h + 1) * HBLK]
            lo16 = lax.shift_right_logical(_rne_hi16(blk[:, :HQ]), 16)
            ref[...] = _rne_hi16(blk[:, HQ:]) | lo16


def _grouped_gemm(bexp, used, xs, wv, wg):
    grid_spec = pltpu.PrefetchScalarGridSpec(
        num_scalar_prefetch=2,
        grid=(NMB,),
        in_specs=[
            pl.BlockSpec((R, D), lambda m, be, us: (m, 0)),
            pl.BlockSpec((1, D, H), lambda m, be, us: (be[m], 0, 0)),
            pl.BlockSpec((1, D, H), lambda m, be, us: (be[m], 0, 0)),
        ],
        out_specs=[
            pl.BlockSpec((R, HQ), lambda m, be, us: (m, 0)),
            pl.BlockSpec((R, HQ), lambda m, be, us: (m, 0)),
        ],
    )
    return pl.pallas_call(
        _gemm_body,
        grid_spec=grid_spec,
        out_shape=(jax.ShapeDtypeStruct((NPAD, HQ), jnp.int32),
                   jax.ShapeDtypeStruct((NPAD, HQ), jnp.int32)),
        compiler_params=pltpu.CompilerParams(
            dimension_semantics=("arbitrary",)),
    )(bexp, used, xs, wv, wg)


@functools.lru_cache(maxsize=1)
def _sc_kernels():
    mesh = plsc.VectorSubcoreMesh(core_axis_name="c", subcore_axis_name="s")

    @functools.partial(
        pl.kernel,
        mesh=mesh,
        out_type=jax.ShapeDtypeStruct((NPAD, D), jnp.float32),
        scratch_types=[
            pltpu.VMEM((TPW, D), jnp.float32),
            pltpu.VMEM((TPW,), jnp.int32),
            pltpu.VMEM((TPW,), jnp.int32),
            pltpu.SemaphoreType.DMA,
        ],
    )
    def scatter(x_hbm, pos0_hbm, pos1_hbm, xs_hbm, rows_v, i0_v, i1_v, sem):
        wid = lax.axis_index("s") * NC + lax.axis_index("c")
        base = wid * TPW
        pltpu.sync_copy(x_hbm.at[pl.ds(base, TPW)], rows_v)
        pltpu.sync_copy(pos0_hbm.at[pl.ds(base, TPW)], i0_v)
        pltpu.sync_copy(pos1_hbm.at[pl.ds(base, TPW)], i1_v)
        c0 = pltpu.async_copy(rows_v, xs_hbm.at[i0_v], sem)
        c1 = pltpu.async_copy(rows_v, xs_hbm.at[i1_v], sem)
        c0.wait()
        c1.wait()

    @functools.partial(
        pl.kernel,
        mesh=mesh,
        out_type=jax.ShapeDtypeStruct((B, H), jnp.float32),
        scratch_types=[
            pltpu.VMEM((2, CH, HQ), jnp.int32),
            pltpu.VMEM((2, CH, HQ), jnp.int32),
            pltpu.VMEM((CH, H), jnp.float32),
            pltpu.VMEM((2, CH), jnp.int32),
            pltpu.VMEM((2, CH), jnp.int32),
            pltpu.VMEM((CH,), jnp.float32),
            pltpu.VMEM((CH,), jnp.float32),
            pltpu.SemaphoreType.DMA,
            pltpu.SemaphoreType.DMA,
        ],
    )
    def combine(ylo_hbm, yhi_hbm, pos0_hbm, pos1_hbm, s0_hbm, s1_hbm, out_hbm,
                ya_v, yb_v, ob_v, ia_v, ib_v, sa_v, sb_v, sem0, sem1):
        wid = lax.axis_index("s") * NC + lax.axis_index("c")
        base = wid * TPW
        NCH = TPW // CH
        halves = (ylo_hbm, yhi_hbm)
        sems = (sem0, sem1)
        steps = [(cc, h) for cc in range(NCH) for h in range(2)]

        def load_idx(cc):
            tb = base + cc * CH
            pltpu.sync_copy(pos0_hbm.at[pl.ds(tb, CH)], ia_v.at[cc % 2])
            pltpu.sync_copy(pos1_hbm.at[pl.ds(tb, CH)], ib_v.at[cc % 2])

        def fire(cc, h, buf):
            sem = sems[buf]
            ca = pltpu.async_copy(halves[h].at[ia_v.at[cc % 2]],
                                  ya_v.at[buf], sem)
            cb = pltpu.async_copy(halves[h].at[ib_v.at[cc % 2]],
                                  yb_v.at[buf], sem)
            return ca, cb

        load_idx(0)
        pend = fire(0, 0, 0)
        for s, (cc, h) in enumerate(steps):
            buf = s % 2
            tb = base + cc * CH
            if h == 0:
                pltpu.sync_copy(s0_hbm.at[pl.ds(tb, CH)], sa_v)
                pltpu.sync_copy(s1_hbm.at[pl.ds(tb, CH)], sb_v)
                sa = sa_v[...]
                sb = sb_v[...]
                sas = [sa[t] for t in range(CH)]
                sbs = [sb[t] for t in range(CH)]
            if s + 1 < len(steps):
                cc2, h2 = steps[s + 1]
                if h2 == 0:
                    load_idx(cc2)
                nxt = fire(cc2, h2, (s + 1) % 2)
            pend[0].wait()
            pend[1].wait()

            def body(hc, _):
                sl = pl.ds(hc * LN, LN)
                for t in range(CH):
                    wa = ya_v[buf, t, sl]
                    wb = yb_v[buf, t, sl]
                    a_lo = lax.bitcast_convert_type(wa << 16, jnp.float32)
                    a_hi = lax.bitcast_convert_type(wa & jnp.int32(-65536),
                                                    jnp.float32)
                    b_lo = lax.bitcast_convert_type(wb << 16, jnp.float32)
                    b_hi = lax.bitcast_convert_type(wb & jnp.int32(-65536),
                                                    jnp.float32)
                    ob_v[t, pl.ds(h * HBLK + hc * LN, LN)] = (
                        sas[t] * a_lo + sbs[t] * b_lo)
                    ob_v[t, pl.ds(h * HBLK + HQ + hc * LN, LN)] = (
                        sas[t] * a_hi + sbs[t] * b_hi)
                return 0

            lax.fori_loop(0, HQ // LN, body, 0)
            if h == 1:
                pltpu.sync_copy(ob_v, out_hbm.at[pl.ds(tb, CH)])
            if s + 1 < len(steps):
                pend = nxt

    return scatter, combine


def kernel(x, expert_weights_v, expert_weights_g, gate_w, gate_b):
    xt = x.T
    gb = gate_b.reshape(E, 1)
    pos0, pos1, s0, s1, bexp, used = _router(xt, gate_w, gb)
    pos0 = pos0.reshape(B)
    pos1 = pos1.reshape(B)
    scatter, combine = _sc_kernels()
    xs = scatter(x, pos0, pos1)
    ys_lo, ys_hi = _grouped_gemm(bexp.reshape(NMB), used.reshape(1),
                                 xs, expert_weights_v, expert_weights_g)
    return combine(ys_lo, ys_hi, pos0, pos1, s0.reshape(B), s1.reshape(B))


# speedup vs baseline: 1.3064x; 1.0566x over previous
import functools

import jax
import jax.numpy as jnp
from jax import lax
from jax.experimental import pallas as pl
from jax.experimental.pallas import tpu as pltpu
from jax.experimental.pallas import tpu_sc as plsc

B, D, H, E = 2048, 1024, 2048, 8
R = 256
NMB = (B * 2) // R + E
NPAD = NMB * R
HBLK = 1024
NHB = H // HBLK
HQ = HBLK // 2

NC, NS = 2, 16
NW = NC * NS
TPW = B // NW
CH = 16
LN = 16


def _cumsum_lanes(a):
    n = 1
    w = a.shape[1]
    while n < w:
        pad = jnp.zeros((a.shape[0], n), a.dtype)
        a = a + jnp.concatenate([pad, a[:, : w - n]], axis=1)
        n *= 2
    return a


def _cumsum_sublanes(a):
    n = 1
    h = a.shape[0]
    while n < h:
        pad = jnp.zeros((n, a.shape[1]), a.dtype)
        a = a + jnp.concatenate([pad, a[: h - n, :]], axis=0)
        n *= 2
    return a


def _router_body(x_ref, gw_ref, gb_ref,
                 pos0_ref, pos1_ref, s0_ref, s1_ref, bexp_ref, used_ref):
    gw = gw_ref[...].astype(jnp.bfloat16)
    xb = x_ref[...].astype(jnp.bfloat16)
    logits = lax.dot_general(
        gw, xb, (((1,), (1,)), ((), ())),
        preferred_element_type=jnp.float32)
    logits = logits + gb_ref[...]
    m = jnp.max(logits, axis=0, keepdims=True)
    p = jnp.exp(logits - m)
    p = p / jnp.sum(p, axis=0, keepdims=True)

    eidx = lax.broadcasted_iota(jnp.int32, (E, B), 0).astype(jnp.float32)
    s1 = jnp.max(p, axis=0, keepdims=True)
    i1 = jnp.min(jnp.where(p == s1, eidx, jnp.float32(E)), axis=0, keepdims=True)
    oh0 = eidx == i1
    p2 = jnp.where(oh0, jnp.float32(-1.0), p)
    s2 = jnp.max(p2, axis=0, keepdims=True)
    i2 = jnp.min(jnp.where(p2 == s2, eidx, jnp.float32(E)), axis=0, keepdims=True)
    oh1 = eidx == i2

    c0 = _cumsum_lanes(oh0.astype(jnp.float32))
    c1 = _cumsum_lanes(oh1.astype(jnp.float32))
    t0 = c0[:, B - 1:B]
    t1 = c1[:, B - 1:B]
    g = t0 + t1
    gp = jnp.floor((g + (R - 1)) / R) * R
    base_incl = _cumsum_sublanes(gp)
    base = base_incl - gp

    pos0 = jnp.sum(jnp.where(oh0, base + c0 - 1.0, 0.0), axis=0, keepdims=True)
    pos1 = jnp.sum(jnp.where(oh1, base + t0 + c1 - 1.0, 0.0), axis=0, keepdims=True)
    pos0_ref[...] = pos0.astype(jnp.int32)
    pos1_ref[...] = pos1.astype(jnp.int32)
    s0_ref[...] = s1
    s1_ref[...] = s2

    total = base_incl[E - 1:E, :]
    usedb = total / R
    used_ref[...] = usedb.astype(jnp.int32)
    bi = lax.broadcasted_iota(jnp.int32, (1, NMB), 1).astype(jnp.float32)
    bi = jnp.minimum(bi, usedb - 1.0)
    bexp = jnp.sum((base <= bi * R).astype(jnp.float32), axis=0, keepdims=True) - 1.0
    bexp_ref[...] = bexp.astype(jnp.int32)


def _router(x, gw, gb):
    return pl.pallas_call(
        _router_body,
        out_shape=(
            jax.ShapeDtypeStruct((1, B), jnp.int32),
            jax.ShapeDtypeStruct((1, B), jnp.int32),
            jax.ShapeDtypeStruct((1, B), jnp.float32),
            jax.ShapeDtypeStruct((1, B), jnp.float32),
            jax.ShapeDtypeStruct((1, NMB), jnp.int32),
            jax.ShapeDtypeStruct((1, 1), jnp.int32),
        ),
    )(x, gw, gb)


def _rne_hi16(a):
    u = lax.bitcast_convert_type(a, jnp.int32)
    r = u + jnp.int32(0x7FFF) + ((u >> 16) & jnp.int32(1))
    return r & jnp.int32(-65536)


def _gemm_body(bexp_ref, used_ref, xs_ref, wv_ref, wg_ref, ylo_ref, yhi_ref):
    m = pl.program_id(0)

    @pl.when(m < used_ref[0])
    def _():
        xb = xs_ref[...]
        wv = wv_ref[0]
        wg = wg_ref[0]
        v = jnp.dot(xb, wv, preferred_element_type=jnp.float32)
        g = jnp.dot(xb, wg, preferred_element_type=jnp.float32)
        act = v * (1.0 / (1.0 + jnp.exp(-g)))
        for h, ref in ((0, ylo_ref), (1, yhi_ref)):
            blk = act[:, h * HBLK:(h + 1) * HBLK]
            lo16 = lax.shift_right_logical(_rne_hi16(blk[:, :HQ]), 16)
            ref[...] = _rne_hi16(blk[:, HQ:]) | lo16


def _grouped_gemm(bexp, used, xs, wv, wg):
    grid_spec = pltpu.PrefetchScalarGridSpec(
        num_scalar_prefetch=2,
        grid=(NMB,),
        in_specs=[
            pl.BlockSpec((R, D), lambda m, be, us: (m, 0)),
            pl.BlockSpec((1, D, H), lambda m, be, us: (be[m], 0, 0)),
            pl.BlockSpec((1, D, H), lambda m, be, us: (be[m], 0, 0)),
        ],
        out_specs=[
            pl.BlockSpec((R, HQ), lambda m, be, us: (m, 0)),
            pl.BlockSpec((R, HQ), lambda m, be, us: (m, 0)),
        ],
    )
    return pl.pallas_call(
        _gemm_body,
        grid_spec=grid_spec,
        out_shape=(jax.ShapeDtypeStruct((NPAD, HQ), jnp.int32),
                   jax.ShapeDtypeStruct((NPAD, HQ), jnp.int32)),
        compiler_params=pltpu.CompilerParams(
            dimension_semantics=("arbitrary",)),
    )(bexp, used, xs, wv, wg)


@functools.lru_cache(maxsize=1)
def _sc_kernels():
    mesh = plsc.VectorSubcoreMesh(core_axis_name="c", subcore_axis_name="s")

    @functools.partial(
        pl.kernel,
        mesh=mesh,
        out_type=jax.ShapeDtypeStruct((NPAD, D), jnp.float32),
        scratch_types=[
            pltpu.VMEM((TPW, D), jnp.float32),
            pltpu.VMEM((TPW,), jnp.int32),
            pltpu.VMEM((TPW,), jnp.int32),
            pltpu.SemaphoreType.DMA,
        ],
    )
    def scatter(x_hbm, pos0_hbm, pos1_hbm, xs_hbm, rows_v, i0_v, i1_v, sem):
        wid = lax.axis_index("s") * NC + lax.axis_index("c")
        base = wid * TPW
        pltpu.sync_copy(x_hbm.at[pl.ds(base, TPW)], rows_v)
        pltpu.sync_copy(pos0_hbm.at[pl.ds(base, TPW)], i0_v)
        pltpu.sync_copy(pos1_hbm.at[pl.ds(base, TPW)], i1_v)
        c0 = pltpu.async_copy(rows_v, xs_hbm.at[i0_v], sem)
        c1 = pltpu.async_copy(rows_v, xs_hbm.at[i1_v], sem)
        c0.wait()
        c1.wait()

    @functools.partial(
        pl.kernel,
        mesh=mesh,
        out_type=jax.ShapeDtypeStruct((B, H), jnp.float32),
        scratch_types=[
            pltpu.VMEM((2, CH, HQ), jnp.int32),
            pltpu.VMEM((2, CH, HQ), jnp.int32),
            pltpu.VMEM((CH, H), jnp.float32),
            pltpu.VMEM((2, CH), jnp.int32),
            pltpu.VMEM((2, CH), jnp.int32),
            pltpu.VMEM((CH,), jnp.float32),
            pltpu.VMEM((CH,), jnp.float32),
            pltpu.SemaphoreType.DMA,
            pltpu.SemaphoreType.DMA,
        ],
    )
    def combine(ylo_hbm, yhi_hbm, pos0_hbm, pos1_hbm, s0_hbm, s1_hbm, out_hbm,
                ya_v, yb_v, ob_v, ia_v, ib_v, sa_v, sb_v, sem0, sem1):
        wid = lax.axis_index("s") * NC + lax.axis_index("c")
        base = wid * TPW
        NCH = TPW // CH
        halves = (ylo_hbm, yhi_hbm)
        sems = (sem0, sem1)
        steps = [(cc, h) for cc in range(NCH) for h in range(2)]

        def load_idx(cc):
            tb = base + cc * CH
            pltpu.sync_copy(pos0_hbm.at[pl.ds(tb, CH)], ia_v.at[cc % 2])
            pltpu.sync_copy(pos1_hbm.at[pl.ds(tb, CH)], ib_v.at[cc % 2])

        def fire(cc, h, buf):
            sem = sems[buf]
            ca = pltpu.async_copy(halves[h].at[ia_v.at[cc % 2]],
                                  ya_v.at[buf], sem)
            cb = pltpu.async_copy(halves[h].at[ib_v.at[cc % 2]],
                                  yb_v.at[buf], sem)
            return ca, cb

        load_idx(0)
        pend = fire(0, 0, 0)
        for s, (cc, h) in enumerate(steps):
            buf = s % 2
            tb = base + cc * CH
            if h == 0:
                pltpu.sync_copy(s0_hbm.at[pl.ds(tb, CH)], sa_v)
                pltpu.sync_copy(s1_hbm.at[pl.ds(tb, CH)], sb_v)
                sa = sa_v[...]
                sb = sb_v[...]
                sas = [sa[t] for t in range(CH)]
                sbs = [sb[t] for t in range(CH)]
            if s + 1 < len(steps):
                cc2, h2 = steps[s + 1]
                if h2 == 0:
                    load_idx(cc2)
                nxt = fire(cc2, h2, (s + 1) % 2)
            pend[0].wait()
            pend[1].wait()

            def body(hc, _):
                sl = pl.ds(hc * LN, LN)
                for t in range(CH):
                    wa = ya_v[buf, t, sl]
                    wb = yb_v[buf, t, sl]
                    a_lo = lax.bitcast_convert_type(wa << 16, jnp.float32)
                    a_hi = lax.bitcast_convert_type(wa & jnp.int32(-65536),
                                                    jnp.float32)
                    b_lo = lax.bitcast_convert_type(wb << 16, jnp.float32)
                    b_hi = lax.bitcast_convert_type(wb & jnp.int32(-65536),
                                                    jnp.float32)
                    ob_v[t, pl.ds(h * HBLK + hc * LN, LN)] = (
                        sas[t] * a_lo + sbs[t] * b_lo)
                    ob_v[t, pl.ds(h * HBLK + HQ + hc * LN, LN)] = (
                        sas[t] * a_hi + sbs[t] * b_hi)
                return 0

            lax.fori_loop(0, HQ // LN, body, 0)
            if h == 1:
                pltpu.sync_copy(ob_v, out_hbm.at[pl.ds(tb, CH)])
            if s + 1 < len(steps):
                pend = nxt

    return scatter, combine


def kernel(x, expert_weights_v, expert_weights_g, gate_w, gate_b):
    gb = gate_b.reshape(E, 1)
    pos0, pos1, s0, s1, bexp, used = _router(x, gate_w, gb)
    pos0 = pos0.reshape(B)
    pos1 = pos1.reshape(B)
    scatter, combine = _sc_kernels()
    xs = scatter(x, pos0, pos1)
    ys_lo, ys_hi = _grouped_gemm(bexp.reshape(NMB), used.reshape(1),
                                 xs, expert_weights_v, expert_weights_g)
    return combine(ys_lo, ys_hi, pos0, pos1, s0.reshape(B), s1.reshape(B))


# speedup vs baseline: 1.3824x; 1.0582x over previous
import functools

import jax
import jax.numpy as jnp
from jax import lax
from jax.experimental import pallas as pl
from jax.experimental.pallas import tpu as pltpu
from jax.experimental.pallas import tpu_sc as plsc

B, D, H, E = 2048, 1024, 2048, 8
R = 256
NMB = (B * 2) // R + E
NPAD = NMB * R
HBLK = 1024
NHB = H // HBLK
HQ = HBLK // 2

NC, NS = 2, 16
NW = NC * NS
TPW = B // NW
CH = 16
LN = 16


def _cumsum_lanes(a):
    n = 1
    w = a.shape[1]
    while n < w:
        pad = jnp.zeros((a.shape[0], n), a.dtype)
        a = a + jnp.concatenate([pad, a[:, : w - n]], axis=1)
        n *= 2
    return a


def _cumsum_sublanes(a):
    n = 1
    h = a.shape[0]
    while n < h:
        pad = jnp.zeros((n, a.shape[1]), a.dtype)
        a = a + jnp.concatenate([pad, a[: h - n, :]], axis=0)
        n *= 2
    return a


def _router_body(x_ref, gw_ref, gb_ref,
                 pos0_ref, pos1_ref, s0_ref, s1_ref, bexp_ref, used_ref):
    gw = gw_ref[...].astype(jnp.bfloat16)
    xb = x_ref[...].astype(jnp.bfloat16)
    logits = lax.dot_general(
        gw, xb, (((1,), (1,)), ((), ())),
        preferred_element_type=jnp.float32)
    logits = logits + gb_ref[...]
    m = jnp.max(logits, axis=0, keepdims=True)
    p = jnp.exp(logits - m)
    p = p / jnp.sum(p, axis=0, keepdims=True)

    eidx = lax.broadcasted_iota(jnp.int32, (E, B), 0).astype(jnp.float32)
    s1 = jnp.max(p, axis=0, keepdims=True)
    i1 = jnp.min(jnp.where(p == s1, eidx, jnp.float32(E)), axis=0, keepdims=True)
    oh0 = eidx == i1
    p2 = jnp.where(oh0, jnp.float32(-1.0), p)
    s2 = jnp.max(p2, axis=0, keepdims=True)
    i2 = jnp.min(jnp.where(p2 == s2, eidx, jnp.float32(E)), axis=0, keepdims=True)
    oh1 = eidx == i2

    c0 = _cumsum_lanes(oh0.astype(jnp.float32))
    c1 = _cumsum_lanes(oh1.astype(jnp.float32))
    t0 = c0[:, B - 1:B]
    t1 = c1[:, B - 1:B]
    g = t0 + t1
    gp = jnp.floor((g + (R - 1)) / R) * R
    base_incl = _cumsum_sublanes(gp)
    base = base_incl - gp

    pos0 = jnp.sum(jnp.where(oh0, base + c0 - 1.0, 0.0), axis=0, keepdims=True)
    pos1 = jnp.sum(jnp.where(oh1, base + t0 + c1 - 1.0, 0.0), axis=0, keepdims=True)
    pos0_ref[...] = pos0.astype(jnp.int32)
    pos1_ref[...] = pos1.astype(jnp.int32)
    s0_ref[...] = s1
    s1_ref[...] = s2

    total = base_incl[E - 1:E, :]
    usedb = total / R
    used_ref[...] = usedb.astype(jnp.int32)
    bi = lax.broadcasted_iota(jnp.int32, (1, NMB), 1).astype(jnp.float32)
    bi = jnp.minimum(bi, usedb - 1.0)
    bexp = jnp.sum((base <= bi * R).astype(jnp.float32), axis=0, keepdims=True) - 1.0
    bexp_ref[...] = bexp.astype(jnp.int32)


def _router(x, gw, gb):
    return pl.pallas_call(
        _router_body,
        out_shape=(
            jax.ShapeDtypeStruct((1, B), jnp.int32),
            jax.ShapeDtypeStruct((1, B), jnp.int32),
            jax.ShapeDtypeStruct((1, B), jnp.float32),
            jax.ShapeDtypeStruct((1, B), jnp.float32),
            jax.ShapeDtypeStruct((1, NMB), jnp.int32),
            jax.ShapeDtypeStruct((1, 1), jnp.int32),
        ),
    )(x, gw, gb)


def _rne_hi16(a):
    u = lax.bitcast_convert_type(a, jnp.int32)
    r = u + jnp.int32(0x7FFF) + ((u >> 16) & jnp.int32(1))
    return r & jnp.int32(-65536)


def _gemm_body(bexp_ref, used_ref, xs_ref, wv_ref, wg_ref, ylo_ref, yhi_ref):
    m = pl.program_id(0)

    @pl.when(m < used_ref[0])
    def _():
        xb = xs_ref[...]
        wv = wv_ref[0]
        wg = wg_ref[0]
        v = jnp.dot(xb, wv, preferred_element_type=jnp.float32)
        g = jnp.dot(xb, wg, preferred_element_type=jnp.float32)
        act = v * (1.0 / (1.0 + jnp.exp(-g)))
        for h, ref in ((0, ylo_ref), (1, yhi_ref)):
            blk = act[:, h * HBLK:(h + 1) * HBLK]
            lo16 = lax.shift_right_logical(_rne_hi16(blk[:, :HQ]), 16)
            ref[...] = _rne_hi16(blk[:, HQ:]) | lo16


def _grouped_gemm(bexp, used, xs, wv, wg):
    grid_spec = pltpu.PrefetchScalarGridSpec(
        num_scalar_prefetch=2,
        grid=(NMB,),
        in_specs=[
            pl.BlockSpec((R, D), lambda m, be, us: (m, 0)),
            pl.BlockSpec((1, D, H), lambda m, be, us: (be[m], 0, 0)),
            pl.BlockSpec((1, D, H), lambda m, be, us: (be[m], 0, 0)),
        ],
        out_specs=[
            pl.BlockSpec((R, HQ), lambda m, be, us: (m, 0)),
            pl.BlockSpec((R, HQ), lambda m, be, us: (m, 0)),
        ],
    )
    return pl.pallas_call(
        _gemm_body,
        grid_spec=grid_spec,
        out_shape=(jax.ShapeDtypeStruct((NPAD, HQ), jnp.int32),
                   jax.ShapeDtypeStruct((NPAD, HQ), jnp.int32)),
        compiler_params=pltpu.CompilerParams(
            dimension_semantics=("arbitrary",)),
    )(bexp, used, xs, wv, wg)


@functools.lru_cache(maxsize=1)
def _sc_kernels():
    mesh = plsc.VectorSubcoreMesh(core_axis_name="c", subcore_axis_name="s")

    @functools.partial(
        pl.kernel,
        mesh=mesh,
        out_type=jax.ShapeDtypeStruct((NPAD, D), jnp.float32),
        scratch_types=[
            pltpu.VMEM((TPW, D), jnp.float32),
            pltpu.VMEM((TPW,), jnp.int32),
            pltpu.VMEM((TPW,), jnp.int32),
            pltpu.SemaphoreType.DMA,
        ],
    )
    def scatter(x_hbm, pos0_hbm, pos1_hbm, xs_hbm, rows_v, i0_v, i1_v, sem):
        wid = lax.axis_index("s") * NC + lax.axis_index("c")
        base = wid * TPW
        l0 = pltpu.async_copy(x_hbm.at[pl.ds(base, TPW)], rows_v, sem)
        l1 = pltpu.async_copy(pos0_hbm.at[pl.ds(base, TPW)], i0_v, sem)
        l2 = pltpu.async_copy(pos1_hbm.at[pl.ds(base, TPW)], i1_v, sem)
        l0.wait()
        l1.wait()
        l2.wait()
        c0 = pltpu.async_copy(rows_v, xs_hbm.at[i0_v], sem)
        c1 = pltpu.async_copy(rows_v, xs_hbm.at[i1_v], sem)
        c0.wait()
        c1.wait()

    @functools.partial(
        pl.kernel,
        mesh=mesh,
        out_type=jax.ShapeDtypeStruct((B, H), jnp.float32),
        scratch_types=[
            pltpu.VMEM((2, CH, HQ), jnp.int32),
            pltpu.VMEM((2, CH, HQ), jnp.int32),
            pltpu.VMEM((2, CH, H), jnp.float32),
            pltpu.VMEM((TPW,), jnp.int32),
            pltpu.VMEM((TPW,), jnp.int32),
            pltpu.VMEM((TPW,), jnp.float32),
            pltpu.VMEM((TPW,), jnp.float32),
            pltpu.SemaphoreType.DMA,
            pltpu.SemaphoreType.DMA,
            pltpu.SemaphoreType.DMA,
        ],
    )
    def combine(ylo_hbm, yhi_hbm, pos0_hbm, pos1_hbm, s0_hbm, s1_hbm, out_hbm,
                ya_v, yb_v, ob_v, ia_v, ib_v, sa_v, sb_v, sem0, sem1, wsem):
        wid = lax.axis_index("s") * NC + lax.axis_index("c")
        base = wid * TPW
        NCH = TPW // CH
        halves = (ylo_hbm, yhi_hbm)
        sems = (sem0, sem1)
        l0 = pltpu.async_copy(pos0_hbm.at[pl.ds(base, TPW)], ia_v, sem0)
        l1 = pltpu.async_copy(pos1_hbm.at[pl.ds(base, TPW)], ib_v, sem0)
        l2 = pltpu.async_copy(s0_hbm.at[pl.ds(base, TPW)], sa_v, sem0)
        l3 = pltpu.async_copy(s1_hbm.at[pl.ds(base, TPW)], sb_v, sem0)
        l0.wait()
        l1.wait()
        l2.wait()
        l3.wait()
        steps = [(cc, h) for cc in range(NCH) for h in range(2)]

        def fire(cc, h, buf):
            ca = pltpu.async_copy(halves[h].at[ia_v.at[pl.ds(cc * CH, CH)]],
                                  ya_v.at[buf], sems[buf])
            cb = pltpu.async_copy(halves[h].at[ib_v.at[pl.ds(cc * CH, CH)]],
                                  yb_v.at[buf], sems[buf])
            return ca, cb

        pend = fire(0, 0, 0)
        wpend = []
        for s, (cc, h) in enumerate(steps):
            buf = s % 2
            op = cc % 2
            tb = base + cc * CH
            if h == 0:
                sa = sa_v[pl.ds(cc * CH, CH)]
                sb = sb_v[pl.ds(cc * CH, CH)]
                sas = [sa[t] for t in range(CH)]
                sbs = [sb[t] for t in range(CH)]
                if cc >= 2:
                    wpend.pop(0).wait()
            if s + 1 < len(steps):
                cc2, h2 = steps[s + 1]
                nxt = fire(cc2, h2, (s + 1) % 2)
            pend[0].wait()
            pend[1].wait()

            def body(hc, _):
                sl = pl.ds(hc * LN, LN)
                for t in range(CH):
                    wa = ya_v[buf, t, sl]
                    wb = yb_v[buf, t, sl]
                    a_lo = lax.bitcast_convert_type(wa << 16, jnp.float32)
                    a_hi = lax.bitcast_convert_type(wa & jnp.int32(-65536),
                                                    jnp.float32)
                    b_lo = lax.bitcast_convert_type(wb << 16, jnp.float32)
                    b_hi = lax.bitcast_convert_type(wb & jnp.int32(-65536),
                                                    jnp.float32)
                    ob_v[op, t, pl.ds(h * HBLK + hc * LN, LN)] = (
                        sas[t] * a_lo + sbs[t] * b_lo)
                    ob_v[op, t, pl.ds(h * HBLK + HQ + hc * LN, LN)] = (
                        sas[t] * a_hi + sbs[t] * b_hi)
                return 0

            lax.fori_loop(0, HQ // LN, body, 0)
            if h == 1:
                wpend.append(pltpu.async_copy(
                    ob_v.at[op], out_hbm.at[pl.ds(tb, CH)], wsem))
            if s + 1 < len(steps):
                pend = nxt
        for w in wpend:
            w.wait()

    return scatter, combine


def kernel(x, expert_weights_v, expert_weights_g, gate_w, gate_b):
    gb = gate_b.reshape(E, 1)
    pos0, pos1, s0, s1, bexp, used = _router(x, gate_w, gb)
    pos0 = pos0.reshape(B)
    pos1 = pos1.reshape(B)
    scatter, combine = _sc_kernels()
    xs = scatter(x, pos0, pos1)
    ys_lo, ys_hi = _grouped_gemm(bexp.reshape(NMB), used.reshape(1),
                                 xs, expert_weights_v, expert_weights_g)
    return combine(ys_lo, ys_hi, pos0, pos1, s0.reshape(B), s1.reshape(B))


# speedup vs baseline: 1.4062x; 1.0172x over previous
import functools

import jax
import jax.numpy as jnp
from jax import lax
from jax.experimental import pallas as pl
from jax.experimental.pallas import tpu as pltpu
from jax.experimental.pallas import tpu_sc as plsc

B, D, H, E = 2048, 1024, 2048, 8
R = 256
NMB = (B * 2) // R + E
NPAD = NMB * R
HBLK = 1024
NHB = H // HBLK
HQ = HBLK // 2

NC, NS = 2, 16
NW = NC * NS
TPW = B // NW
CH = 16
LN = 16


def _cumsum_lanes(a):
    n = 1
    w = a.shape[1]
    while n < w:
        pad = jnp.zeros((a.shape[0], n), a.dtype)
        a = a + jnp.concatenate([pad, a[:, : w - n]], axis=1)
        n *= 2
    return a


def _cumsum_sublanes(a):
    n = 1
    h = a.shape[0]
    while n < h:
        pad = jnp.zeros((n, a.shape[1]), a.dtype)
        a = a + jnp.concatenate([pad, a[: h - n, :]], axis=0)
        n *= 2
    return a


def _router_body(x_ref, gw_ref, gb_ref,
                 pos0_ref, pos1_ref, s0_ref, s1_ref, bexp_ref, used_ref):
    gw = gw_ref[...].astype(jnp.bfloat16)
    xb = x_ref[...].astype(jnp.bfloat16)
    logits = lax.dot_general(
        gw, xb, (((1,), (1,)), ((), ())),
        preferred_element_type=jnp.float32)
    logits = logits + gb_ref[...]
    m = jnp.max(logits, axis=0, keepdims=True)
    p = jnp.exp(logits - m)
    p = p / jnp.sum(p, axis=0, keepdims=True)

    eidx = lax.broadcasted_iota(jnp.int32, (E, B), 0).astype(jnp.float32)
    s1 = jnp.max(p, axis=0, keepdims=True)
    i1 = jnp.min(jnp.where(p == s1, eidx, jnp.float32(E)), axis=0, keepdims=True)
    oh0 = eidx == i1
    p2 = jnp.where(oh0, jnp.float32(-1.0), p)
    s2 = jnp.max(p2, axis=0, keepdims=True)
    i2 = jnp.min(jnp.where(p2 == s2, eidx, jnp.float32(E)), axis=0, keepdims=True)
    oh1 = eidx == i2

    c0 = _cumsum_lanes(oh0.astype(jnp.float32))
    c1 = _cumsum_lanes(oh1.astype(jnp.float32))
    t0 = c0[:, B - 1:B]
    t1 = c1[:, B - 1:B]
    g = t0 + t1
    gp = jnp.floor((g + (R - 1)) / R) * R
    base_incl = _cumsum_sublanes(gp)
    base = base_incl - gp

    pos0 = jnp.sum(jnp.where(oh0, base + c0 - 1.0, 0.0), axis=0, keepdims=True)
    pos1 = jnp.sum(jnp.where(oh1, base + t0 + c1 - 1.0, 0.0), axis=0, keepdims=True)
    pos0_ref[...] = pos0.astype(jnp.int32)
    pos1_ref[...] = pos1.astype(jnp.int32)
    s0_ref[...] = s1
    s1_ref[...] = s2

    total = base_incl[E - 1:E, :]
    usedb = total / R
    used_ref[...] = usedb.astype(jnp.int32)
    bi = lax.broadcasted_iota(jnp.int32, (1, NMB), 1).astype(jnp.float32)
    bi = jnp.minimum(bi, usedb - 1.0)
    bexp = jnp.sum((base <= bi * R).astype(jnp.float32), axis=0, keepdims=True) - 1.0
    bexp_ref[...] = bexp.astype(jnp.int32)


def _router(x, gw, gb):
    return pl.pallas_call(
        _router_body,
        out_shape=(
            jax.ShapeDtypeStruct((1, B), jnp.int32),
            jax.ShapeDtypeStruct((1, B), jnp.int32),
            jax.ShapeDtypeStruct((1, B), jnp.float32),
            jax.ShapeDtypeStruct((1, B), jnp.float32),
            jax.ShapeDtypeStruct((1, NMB), jnp.int32),
            jax.ShapeDtypeStruct((1, 1), jnp.int32),
        ),
    )(x, gw, gb)


def _rne_hi16(a):
    u = lax.bitcast_convert_type(a, jnp.int32)
    r = u + jnp.int32(0x7FFF) + ((u >> 16) & jnp.int32(1))
    return r & jnp.int32(-65536)


def _gemm_body(bexp_ref, used_ref, xs_ref, wv_ref, wg_ref, ylo_ref, yhi_ref):
    m = pl.program_id(0)

    @pl.when(m < used_ref[0])
    def _():
        xb = xs_ref[...]
        wv = wv_ref[0]
        wg = wg_ref[0]
        v = jnp.dot(xb, wv, preferred_element_type=jnp.float32)
        g = jnp.dot(xb, wg, preferred_element_type=jnp.float32)
        act = v * (1.0 / (1.0 + jnp.exp(-g)))
        for h, ref in ((0, ylo_ref), (1, yhi_ref)):
            blk = act[:, h * HBLK:(h + 1) * HBLK]
            lo16 = lax.shift_right_logical(_rne_hi16(blk[:, :HQ]), 16)
            ref[...] = _rne_hi16(blk[:, HQ:]) | lo16


def _grouped_gemm(bexp, used, xs, wv, wg):
    grid_spec = pltpu.PrefetchScalarGridSpec(
        num_scalar_prefetch=2,
        grid=(NMB,),
        in_specs=[
            pl.BlockSpec((R, D), lambda m, be, us: (jnp.minimum(m, us[0] - 1), 0)),
            pl.BlockSpec((1, D, H), lambda m, be, us: (be[m], 0, 0)),
            pl.BlockSpec((1, D, H), lambda m, be, us: (be[m], 0, 0)),
        ],
        out_specs=[
            pl.BlockSpec((R, HQ), lambda m, be, us: (m, 0)),
            pl.BlockSpec((R, HQ), lambda m, be, us: (m, 0)),
        ],
    )
    return pl.pallas_call(
        _gemm_body,
        grid_spec=grid_spec,
        out_shape=(jax.ShapeDtypeStruct((NPAD, HQ), jnp.int32),
                   jax.ShapeDtypeStruct((NPAD, HQ), jnp.int32)),
        compiler_params=pltpu.CompilerParams(
            dimension_semantics=("arbitrary",)),
    )(bexp, used, xs, wv, wg)


@functools.lru_cache(maxsize=1)
def _sc_kernels():
    mesh = plsc.VectorSubcoreMesh(core_axis_name="c", subcore_axis_name="s")

    @functools.partial(
        pl.kernel,
        mesh=mesh,
        out_type=jax.ShapeDtypeStruct((NPAD, D), jnp.float32),
        scratch_types=[
            pltpu.VMEM((TPW, D), jnp.float32),
            pltpu.VMEM((TPW,), jnp.int32),
            pltpu.VMEM((TPW,), jnp.int32),
            pltpu.SemaphoreType.DMA,
        ],
    )
    def scatter(x_hbm, pos0_hbm, pos1_hbm, xs_hbm, rows_v, i0_v, i1_v, sem):
        wid = lax.axis_index("s") * NC + lax.axis_index("c")
        base = wid * TPW
        l0 = pltpu.async_copy(x_hbm.at[pl.ds(base, TPW)], rows_v, sem)
        l1 = pltpu.async_copy(pos0_hbm.at[pl.ds(base, TPW)], i0_v, sem)
        l2 = pltpu.async_copy(pos1_hbm.at[pl.ds(base, TPW)], i1_v, sem)
        l0.wait()
        l1.wait()
        l2.wait()
        c0 = pltpu.async_copy(rows_v, xs_hbm.at[i0_v], sem)
        c1 = pltpu.async_copy(rows_v, xs_hbm.at[i1_v], sem)
        c0.wait()
        c1.wait()

    @functools.partial(
        pl.kernel,
        mesh=mesh,
        out_type=jax.ShapeDtypeStruct((B, H), jnp.float32),
        scratch_types=[
            pltpu.VMEM((2, CH, HQ), jnp.int32),
            pltpu.VMEM((2, CH, HQ), jnp.int32),
            pltpu.VMEM((2, CH, H), jnp.float32),
            pltpu.VMEM((TPW,), jnp.int32),
            pltpu.VMEM((TPW,), jnp.int32),
            pltpu.VMEM((TPW,), jnp.float32),
            pltpu.VMEM((TPW,), jnp.float32),
            pltpu.SemaphoreType.DMA,
            pltpu.SemaphoreType.DMA,
            pltpu.SemaphoreType.DMA,
        ],
    )
    def combine(ylo_hbm, yhi_hbm, pos0_hbm, pos1_hbm, s0_hbm, s1_hbm, out_hbm,
                ya_v, yb_v, ob_v, ia_v, ib_v, sa_v, sb_v, sem0, sem1, wsem):
        wid = lax.axis_index("s") * NC + lax.axis_index("c")
        base = wid * TPW
        NCH = TPW // CH
        halves = (ylo_hbm, yhi_hbm)
        sems = (sem0, sem1)
        l0 = pltpu.async_copy(pos0_hbm.at[pl.ds(base, TPW)], ia_v, sem0)
        l1 = pltpu.async_copy(pos1_hbm.at[pl.ds(base, TPW)], ib_v, sem0)
        l2 = pltpu.async_copy(s0_hbm.at[pl.ds(base, TPW)], sa_v, sem0)
        l3 = pltpu.async_copy(s1_hbm.at[pl.ds(base, TPW)], sb_v, sem0)
        l0.wait()
        l1.wait()
        l2.wait()
        l3.wait()
        steps = [(cc, h) for cc in range(NCH) for h in range(2)]

        def fire(cc, h, buf):
            ca = pltpu.async_copy(halves[h].at[ia_v.at[pl.ds(cc * CH, CH)]],
                                  ya_v.at[buf], sems[buf])
            cb = pltpu.async_copy(halves[h].at[ib_v.at[pl.ds(cc * CH, CH)]],
                                  yb_v.at[buf], sems[buf])
            return ca, cb

        pend = fire(0, 0, 0)
        wpend = []
        for s, (cc, h) in enumerate(steps):
            buf = s % 2
            op = cc % 2
            tb = base + cc * CH
            if h == 0:
                sa = sa_v[pl.ds(cc * CH, CH)]
                sb = sb_v[pl.ds(cc * CH, CH)]
                sas = [sa[t] for t in range(CH)]
                sbs = [sb[t] for t in range(CH)]
                if cc >= 2:
                    wpend.pop(0).wait()
            if s + 1 < len(steps):
                cc2, h2 = steps[s + 1]
                nxt = fire(cc2, h2, (s + 1) % 2)
            pend[0].wait()
            pend[1].wait()

            def body(hc, _):
                sl = pl.ds(hc * LN, LN)
                for t in range(CH):
                    wa = ya_v[buf, t, sl]
                    wb = yb_v[buf, t, sl]
                    a_lo = lax.bitcast_convert_type(wa << 16, jnp.float32)
                    a_hi = lax.bitcast_convert_type(wa & jnp.int32(-65536),
                                                    jnp.float32)
                    b_lo = lax.bitcast_convert_type(wb << 16, jnp.float32)
                    b_hi = lax.bitcast_convert_type(wb & jnp.int32(-65536),
                                                    jnp.float32)
                    ob_v[op, t, pl.ds(h * HBLK + hc * LN, LN)] = (
                        sas[t] * a_lo + sbs[t] * b_lo)
                    ob_v[op, t, pl.ds(h * HBLK + HQ + hc * LN, LN)] = (
                        sas[t] * a_hi + sbs[t] * b_hi)
                return 0

            lax.fori_loop(0, HQ // LN, body, 0)
            if h == 1:
                wpend.append(pltpu.async_copy(
                    ob_v.at[op], out_hbm.at[pl.ds(tb, CH)], wsem))
            if s + 1 < len(steps):
                pend = nxt
        for w in wpend:
            w.wait()

    return scatter, combine


def kernel(x, expert_weights_v, expert_weights_g, gate_w, gate_b):
    gb = gate_b.reshape(E, 1)
    pos0, pos1, s0, s1, bexp, used = _router(x, gate_w, gb)
    pos0 = pos0.reshape(B)
    pos1 = pos1.reshape(B)
    scatter, combine = _sc_kernels()
    xs = scatter(x, pos0, pos1)
    ys_lo, ys_hi = _grouped_gemm(bexp.reshape(NMB), used.reshape(1),
                                 xs, expert_weights_v, expert_weights_g)
    return combine(ys_lo, ys_hi, pos0, pos1, s0.reshape(B), s1.reshape(B))
